# baseline (speedup 1.0000x reference)
import functools

import jax
import jax.numpy as jnp
from jax import lax
from jax.experimental import pallas as pl
from jax.experimental.pallas import tpu as pltpu
from jax.experimental.pallas import tpu_sc as plsc

NC = 2
NS = 16
LN = 16
HALF = 32
CH = 128
DR = 125

_U, _I, _NB = 20000, 20000, 10000


def _mesh():
  return plsc.VectorSubcoreMesh(
      core_axis_name="c", subcore_axis_name="s", num_cores=NC, num_subcores=NS)


def _sds(shape, dtype=jnp.float32):
  return jax.ShapeDtypeStruct(shape, dtype)


NBUF = 4
LOOK = 3


def _make_spmm(n_src, n_out, seg_len, n_seg, mode, scale=1.0):
  rpt = n_out // NS
  assert rpt % DR == 0 and seg_len % NBUF == 0

  def body(*refs):
    if mode == "plain":
      (table, idxp, valsp, out,
       acc, ibig, vbig, rows_buf, sbuf, d1, d2, d1b, d2b, *sems) = refs
      add_into = x_next = None
    elif mode == "first":
      (table, idxp, valsp, add_into, out, x_next,
       acc, ibig, vbig, rows_buf, sbuf, d1, d2, d1b, d2b, *sems) = refs
    else:
      (table, idxp, valsp, add_into, out,
       acc, ibig, vbig, rows_buf, sbuf, d1, d2, d1b, d2b, *sems) = refs
      x_next = None
    gsems, ssems = sems[:NBUF], sems[NBUF:]

    c = lax.axis_index("c")
    s = lax.axis_index("s")
    zvec = jnp.zeros((LN,), jnp.float32)

    def zrow(r, _):
      d1[r, pl.ds(0, LN)] = zvec
      d1[r, pl.ds(LN, LN)] = zvec
      return _
    lax.fori_loop(0, DR, zrow, 0)

    def zchunk(t, _):
      pltpu.sync_copy(d1, acc.at[pl.ds(s * rpt + t * DR, DR)])
      return _
    lax.fori_loop(0, rpt // DR, zchunk, 0)
    plsc.subcore_barrier()

    def gwait(j, k):
      pltpu.make_async_copy(table.at[ibig.at[j, 0]], rows_buf.at[k],
                            gsems[k]).wait()

    def gstart(j, k):
      pltpu.async_copy(table.at[ibig.at[j, 0]], rows_buf.at[k], gsems[k])

    def sstart(j, kp):
      pltpu.async_copy(sbuf.at[kp], acc.at[ibig.at[j, 1]], ssems[kp],
                       add=True)

    def swait(j, kp):
      pltpu.make_async_copy(sbuf.at[kp], acc.at[ibig.at[j, 1]],
                            ssems[kp]).wait()

    def segment(seg, _):
      pltpu.sync_copy(idxp.at[c, s, seg], ibig)
      pltpu.sync_copy(valsp.at[s, seg], vbig)
      for k in range(LOOK):
        gstart(k, k)

      def round_(g, _):
        for k in range(NBUF):
          j = g * NBUF + k
          kp = k % 2
          gwait(j, k)

          @pl.when(j + LOOK < seg_len)
          def _issue_next():
            gstart(j + LOOK, (k + LOOK) % NBUF)

          @pl.when(j >= 2)
          def _wait_prev():
            swait(j - 2, kp)

          vbase = jnp.full((LN,), j * CH, jnp.int32)

          def scl(e, _):
            v = plsc.load_gather(vbig, [vbase + e])
            a, b = plsc.unpack(rows_buf[k, e, pl.ds(0, 2 * LN)],
                               format=plsc.PackFormat.INTERLEAVED)
            sbuf[kp, e, pl.ds(0, LN)] = a * v
            sbuf[kp, e, pl.ds(LN, LN)] = b * v
            return _
          lax.fori_loop(0, CH, scl, 0, unroll=8)

          sstart(j, kp)
        return _
      lax.fori_loop(0, seg_len // NBUF, round_, 0)
      for kp in range(2):
        swait(0, kp)
      return _
    lax.fori_loop(0, n_seg, segment, 0)
    plsc.subcore_barrier()

    def pack_rows(src_f32):
      def pk(r, _):
        d1b[r, pl.ds(0, 2 * LN)] = plsc.pack(
            src_f32[r, pl.ds(0, LN)], src_f32[r, pl.ds(LN, LN)],
            format=plsc.PackFormat.INTERLEAVED)
        return _
      lax.fori_loop(0, DR, pk, 0, unroll=4)

    def dchunk(t, _):
      r0 = s * rpt + t * DR
      g0 = c * n_out + r0
      pltpu.sync_copy(acc.at[pl.ds(r0, DR)], d1)
      if mode == "first":
        pack_rows(d1)
        pltpu.sync_copy(d1b, x_next.at[pl.ds(g0, DR)])
        pltpu.sync_copy(add_into.at[pl.ds(g0, DR)], d2b)

        def addv(r, _):
          a, b = plsc.unpack(d2b[r, pl.ds(0, 2 * LN)],
                             format=plsc.PackFormat.INTERLEAVED)
          d1[r, pl.ds(0, LN)] = d1[r, pl.ds(0, LN)] + a
          d1[r, pl.ds(LN, LN)] = d1[r, pl.ds(LN, LN)] + b
          return _
        lax.fori_loop(0, DR, addv, 0, unroll=4)
        pltpu.sync_copy(d1, out.at[pl.ds(g0, DR)])
      elif mode == "last":
        pltpu.sync_copy(add_into.at[pl.ds(g0, DR)], d2)

        def addv(r, _):
          d1[r, pl.ds(0, LN)] = (d1[r, pl.ds(0, LN)] +
                                 d2[r, pl.ds(0, LN)]) * scale
          d1[r, pl.ds(LN, LN)] = (d1[r, pl.ds(LN, LN)] +
                                  d2[r, pl.ds(LN, LN)]) * scale
          return _
        lax.fori_loop(0, DR, addv, 0, unroll=4)
        pack_rows(d1)
        pltpu.sync_copy(d1b, out.at[pl.ds(g0, DR)])
      else:
        pack_rows(d1)
        pltpu.sync_copy(d1b, out.at[pl.ds(g0, DR)])
      return _
    lax.fori_loop(0, rpt // DR, dchunk, 0)

  if mode == "first":
    outs = (_sds((NC * n_out, HALF)),
            _sds((NC * n_out, HALF), jnp.bfloat16))
  elif mode == "last":
    outs = _sds((NC * n_out, HALF), jnp.bfloat16)
  else:
    outs = _sds((NC * n_out, HALF), jnp.bfloat16)
  return pl.kernel(
      body,
      out_type=outs,
      mesh=_mesh(),
      compiler_params=pltpu.CompilerParams(use_tc_tiling_on_sc=False, needs_layout_passes=False),
      scratch_types=[
          pltpu.VMEM_SHARED((n_out, HALF), jnp.float32),
          pltpu.VMEM((seg_len, 2, CH), jnp.int32),
          pltpu.VMEM((seg_len * CH,), jnp.float32),
          pltpu.VMEM((NBUF, CH, 2 * LN), jnp.bfloat16),
          pltpu.VMEM((2, CH, HALF), jnp.float32),
          pltpu.VMEM((DR, HALF), jnp.float32),
          pltpu.VMEM((DR, HALF), jnp.float32),
          pltpu.VMEM((DR, 2 * LN), jnp.bfloat16),
          pltpu.VMEM((DR, 2 * LN), jnp.bfloat16),
      ] + [pltpu.SemaphoreType.DMA] * (NBUF + 2),
  )


def _make_score(batch):
  pb = batch // NS

  def body(t_ui, t_ub, t_bi, uidx_ui, uidx_ub, bidx_bi, bidx_ub, pred,
           ua, bb, g1, g2, g3, g4, ibuf, pbuf, sem):
    c = lax.axis_index("c")
    s = lax.axis_index("s")
    iota = lax.iota(jnp.int32, LN)

    def gath(idx_hbm, table, dst, n128):
      def g(j, _):
        pltpu.sync_copy(idx_hbm.at[c, pl.ds(s * n128 * CH + j * CH, CH)], ibuf)
        pltpu.async_copy(table.at[ibuf], dst.at[pl.ds(j * CH, CH)], sem).wait()
        return _
      lax.fori_loop(0, n128, g, 0)

    gath(uidx_ui, t_ui, g1, pb // CH)
    gath(uidx_ub, t_ub, g2, pb // CH)
    gath(bidx_bi, t_bi, g3, 2 * pb // CH)
    gath(bidx_ub, t_ub, g4, 2 * pb // CH)

    def upadd(dst, sa, sb, n):
      def go(r, _):
        a1, b1 = plsc.unpack(sa[r, pl.ds(0, 2 * LN)],
                             format=plsc.PackFormat.INTERLEAVED)
        a2, b2 = plsc.unpack(sb[r, pl.ds(0, 2 * LN)],
                             format=plsc.PackFormat.INTERLEAVED)
        dst[r, pl.ds(0, LN)] = a1 + a2
        dst[r, pl.ds(LN, LN)] = b1 + b2
        return _
      lax.fori_loop(0, n, go, 0, unroll=4)

    upadd(ua, g1, g2, pb)
    upadd(bb, g3, g4, 2 * pb)

    def dots(g, _):
      ru = g * LN + iota
      for k in range(2):
        rb = ru * 2 + k
        d = jnp.zeros((LN,), jnp.float32)
        for j in range(HALF):
          cj = jnp.full((LN,), j, jnp.int32)
          a = plsc.load_gather(ua, [ru, cj])
          b = plsc.load_gather(bb, [rb, cj])
          d = d + a * b
        pbuf[k, pl.ds(g * LN, LN)] = d
      return _
    lax.fori_loop(0, pb // LN, dots, 0)

    def wout(k, _):
      pltpu.sync_copy(pbuf.at[k], pred.at[c * 2 + k, pl.ds(s * pb, pb)])
      return _
    lax.fori_loop(0, 2, wout, 0)

  return pl.kernel(
      body,
      out_type=_sds((NC * 2, batch)),
      mesh=_mesh(),
      compiler_params=pltpu.CompilerParams(use_tc_tiling_on_sc=False, needs_layout_passes=False),
      scratch_types=[
          pltpu.VMEM((pb, HALF), jnp.float32),
          pltpu.VMEM((2 * pb, HALF), jnp.float32),
          pltpu.VMEM((pb, 2 * LN), jnp.bfloat16),
          pltpu.VMEM((pb, 2 * LN), jnp.bfloat16),
          pltpu.VMEM((2 * pb, 2 * LN), jnp.bfloat16),
          pltpu.VMEM((2 * pb, 2 * LN), jnp.bfloat16),
          pltpu.VMEM((CH,), jnp.int32),
          pltpu.VMEM((2, pb), jnp.float32),
          pltpu.SemaphoreType.DMA,
      ],
  )


def _split_cols(x):
  n = x.shape[0]
  return x.reshape(n, NC, HALF).transpose(1, 0, 2).reshape(NC * n, HALF)


def _pack_edges(src, dst, vals, n_src, seg_len, n_seg):
  e = src.shape[0]
  n_chunks = seg_len * n_seg
  pad = NS * CH * n_chunks - e
  assert pad >= 0
  if pad:
    src = jnp.pad(src, (0, pad))
    dst = jnp.pad(dst, (0, pad))
    vals = jnp.pad(vals, (0, pad))
  sd = jnp.stack([src, dst], axis=0).reshape(2, NS, n_chunks, CH)
  idxp = jnp.stack(
      [sd.at[0].add(c * n_src) for c in range(NC)],
      axis=0).transpose(0, 2, 3, 1, 4)
  idxp = idxp.reshape(NC, NS, n_seg, seg_len, 2, CH)
  valsp = vals.reshape(NS, n_seg, seg_len * CH)
  return idxp, valsp


def kernel(users_feature, items_feature, bundles_feature,
           ui_rows, ui_cols, ui_vals,
           bi_rows, bi_cols, bi_vals,
           ub_rows, ub_cols, ub_vals,
           users_idx, bundles_idx):
  u, i, nb = _U, _I, _NB
  n_ui = u + i
  n_ub = u + nb

  ui_rows = ui_rows.astype(jnp.int32)
  ui_cols = ui_cols.astype(jnp.int32)
  bi_rows = bi_rows.astype(jnp.int32)
  bi_cols = bi_cols.astype(jnp.int32)
  ub_rows = ub_rows.astype(jnp.int32)
  ub_cols = ub_cols.astype(jnp.int32)

  x0_ui = _split_cols(
      jnp.concatenate([users_feature, items_feature], axis=0)
  ).astype(jnp.bfloat16)
  x0_ub = _split_cols(
      jnp.concatenate([users_feature, bundles_feature], axis=0)
  ).astype(jnp.bfloat16)

  ui_seg, ui_nseg = 40, 16
  bi_seg, bi_nseg = 160, 1
  ub_seg, ub_nseg = 28, 7

  ui_idx, ui_v = _pack_edges(
      jnp.concatenate([ui_rows, ui_cols + u]),
      jnp.concatenate([ui_cols + u, ui_rows]),
      jnp.concatenate([ui_vals, ui_vals]), n_ui, ui_seg, ui_nseg)
  ub_idx, ub_v = _pack_edges(
      jnp.concatenate([ub_rows, ub_cols + u]),
      jnp.concatenate([ub_cols + u, ub_rows]),
      jnp.concatenate([ub_vals, ub_vals]), n_ub, ub_seg, ub_nseg)
  bi_idx, bi_v = _pack_edges(bi_cols + u, bi_rows, bi_vals, n_ui,
                             bi_seg, bi_nseg)

  out1_ui, x1_ui = _make_spmm(n_ui, n_ui, ui_seg, ui_nseg, "first")(
      x0_ui, ui_idx, ui_v, x0_ui)
  out_ui = _make_spmm(n_ui, n_ui, ui_seg, ui_nseg, "last", 1.0 / 3.0)(
      x1_ui, ui_idx, ui_v, out1_ui)
  out_bi = _make_spmm(n_ui, nb, bi_seg, bi_nseg, "plain")(out_ui, bi_idx, bi_v)
  out1_ub, x1_ub = _make_spmm(n_ub, n_ub, ub_seg, ub_nseg, "first")(
      x0_ub, ub_idx, ub_v, x0_ub)
  out_ub = _make_spmm(n_ub, n_ub, ub_seg, ub_nseg, "last", 1.0 / 3.0)(
      x1_ub, ub_idx, ub_v, out1_ub)

  batch = users_idx.shape[0]
  uix = users_idx.astype(jnp.int32)
  bix = bundles_idx.astype(jnp.int32).reshape(-1)
  uidx_ui = jnp.stack([uix + c * n_ui for c in range(NC)], axis=0)
  uidx_ub = jnp.stack([uix + c * n_ub for c in range(NC)], axis=0)
  bidx_bi = jnp.stack([bix + c * nb for c in range(NC)], axis=0)
  bidx_ub = jnp.stack([bix + u + c * n_ub for c in range(NC)], axis=0)

  pred4 = _make_score(batch)(
      out_ui, out_ub, out_bi, uidx_ui, uidx_ub, bidx_bi, bidx_ub)
  return jnp.stack([pred4[0] + pred4[2], pred4[1] + pred4[3]], axis=-1)


# speedup vs baseline: 1.0026x; 1.0026x over previous
import functools

import jax
import jax.numpy as jnp
from jax import lax
from jax.experimental import pallas as pl
from jax.experimental.pallas import tpu as pltpu
from jax.experimental.pallas import tpu_sc as plsc

NC = 2
NS = 16
LN = 16
HALF = 32
CH = 128
DR = 125

_U, _I, _NB = 20000, 20000, 10000


def _mesh():
  return plsc.VectorSubcoreMesh(
      core_axis_name="c", subcore_axis_name="s", num_cores=NC, num_subcores=NS)


def _sds(shape, dtype=jnp.float32):
  return jax.ShapeDtypeStruct(shape, dtype)


NBUF = 4
LOOK = 3


def _phase(c, s, scr, table, idxp, valsp, add_into, out, x_next,
           n_out, seg_len, n_seg, mode, scale=1.0):
  (acc, ibig, vbig, rows_buf, sbuf, d1, d2, d1b, d2b, gsems, ssems) = scr
  rpt = n_out // NS
  assert rpt % DR == 0 and seg_len % NBUF == 0
  zvec = jnp.zeros((LN,), jnp.float32)

  def zrow(r, _):
    d1[r, pl.ds(0, LN)] = zvec
    d1[r, pl.ds(LN, LN)] = zvec
    return _
  lax.fori_loop(0, DR, zrow, 0)

  def zchunk(t, _):
    pltpu.sync_copy(d1, acc.at[pl.ds(s * rpt + t * DR, DR)])
    return _
  lax.fori_loop(0, rpt // DR, zchunk, 0)
  plsc.subcore_barrier()

  def gwait(j, k):
    pltpu.make_async_copy(table.at[ibig.at[j, 0]], rows_buf.at[k],
                          gsems[k]).wait()

  def gstart(j, k):
    pltpu.async_copy(table.at[ibig.at[j, 0]], rows_buf.at[k], gsems[k])

  def sstart(j, kp):
    pltpu.async_copy(sbuf.at[kp], acc.at[ibig.at[j, 1]], ssems[kp],
                     add=True)

  def swait(j, kp):
    pltpu.make_async_copy(sbuf.at[kp], acc.at[ibig.at[j, 1]],
                          ssems[kp]).wait()

  def segment(seg, _):
    pltpu.sync_copy(idxp.at[c, s, seg], ibig)
    pltpu.sync_copy(valsp.at[s, seg], vbig)
    for k in range(LOOK):
      gstart(k, k)

    def round_(g, _):
      for k in range(NBUF):
        j = g * NBUF + k
        kp = k % 2
        gwait(j, k)

        @pl.when(j + LOOK < seg_len)
        def _issue_next():
          gstart(j + LOOK, (k + LOOK) % NBUF)

        @pl.when(j >= 2)
        def _wait_prev():
          swait(j - 2, kp)

        vbase = jnp.full((LN,), j * CH, jnp.int32)

        def scl(e, _):
          v = plsc.load_gather(vbig, [vbase + e])
          a, b = plsc.unpack(rows_buf[k, e, pl.ds(0, 2 * LN)],
                             format=plsc.PackFormat.INTERLEAVED)
          sbuf[kp, e, pl.ds(0, LN)] = a * v
          sbuf[kp, e, pl.ds(LN, LN)] = b * v
          return _
        lax.fori_loop(0, CH, scl, 0, unroll=8)

        sstart(j, kp)
      return _
    lax.fori_loop(0, seg_len // NBUF, round_, 0)
    for kp in range(2):
      swait(0, kp)
    return _
  lax.fori_loop(0, n_seg, segment, 0)
  plsc.subcore_barrier()

  def pack_rows(src_f32):
    def pk(r, _):
      d1b[r, pl.ds(0, 2 * LN)] = plsc.pack(
          src_f32[r, pl.ds(0, LN)], src_f32[r, pl.ds(LN, LN)],
          format=plsc.PackFormat.INTERLEAVED)
      return _
    lax.fori_loop(0, DR, pk, 0, unroll=4)

  def dchunk(t, _):
    r0 = s * rpt + t * DR
    g0 = c * n_out + r0
    pltpu.sync_copy(acc.at[pl.ds(r0, DR)], d1)
    if mode == "first":
      pack_rows(d1)
      pltpu.sync_copy(d1b, x_next.at[pl.ds(g0, DR)])
      pltpu.sync_copy(add_into.at[pl.ds(g0, DR)], d2b)

      def addv(r, _):
        a, b = plsc.unpack(d2b[r, pl.ds(0, 2 * LN)],
                           format=plsc.PackFormat.INTERLEAVED)
        d1[r, pl.ds(0, LN)] = d1[r, pl.ds(0, LN)] + a
        d1[r, pl.ds(LN, LN)] = d1[r, pl.ds(LN, LN)] + b
        return _
      lax.fori_loop(0, DR, addv, 0, unroll=4)
      pltpu.sync_copy(d1, out.at[pl.ds(g0, DR)])
    elif mode == "last":
      pltpu.sync_copy(add_into.at[pl.ds(g0, DR)], d2)

      def addv(r, _):
        d1[r, pl.ds(0, LN)] = (d1[r, pl.ds(0, LN)] +
                               d2[r, pl.ds(0, LN)]) * scale
        d1[r, pl.ds(LN, LN)] = (d1[r, pl.ds(LN, LN)] +
                                d2[r, pl.ds(LN, LN)]) * scale
        return _
      lax.fori_loop(0, DR, addv, 0, unroll=4)
      pack_rows(d1)
      pltpu.sync_copy(d1b, out.at[pl.ds(g0, DR)])
    else:
      pack_rows(d1)
      pltpu.sync_copy(d1b, out.at[pl.ds(g0, DR)])
    return _
  lax.fori_loop(0, rpt // DR, dchunk, 0)
  plsc.subcore_barrier()


def _scratch(n_acc, seg_len):
  return [
      pltpu.VMEM_SHARED((n_acc, HALF), jnp.float32),
      pltpu.VMEM((seg_len, 2, CH), jnp.int32),
      pltpu.VMEM((seg_len * CH,), jnp.float32),
      pltpu.VMEM((NBUF, CH, 2 * LN), jnp.bfloat16),
      pltpu.VMEM((2, CH, HALF), jnp.float32),
      pltpu.VMEM((DR, HALF), jnp.float32),
      pltpu.VMEM((DR, HALF), jnp.float32),
      pltpu.VMEM((DR, 2 * LN), jnp.bfloat16),
      pltpu.VMEM((DR, 2 * LN), jnp.bfloat16),
  ] + [pltpu.SemaphoreType.DMA] * (NBUF + 2)


def _grab(refs, n_io):
  scr = list(refs[n_io:])
  return tuple(scr[:9]) + (scr[9:9 + NBUF], scr[9 + NBUF:])


def _make_k1(n_ui, seg_len, n_seg):
  def body(*refs):
    (x0, idxp, valsp, out1, x1, out_ui) = refs[:6]
    scr = _grab(refs, 6)
    c = lax.axis_index("c")
    s = lax.axis_index("s")
    _phase(c, s, scr, x0, idxp, valsp, x0, out1, x1,
           n_ui, seg_len, n_seg, "first")
    _phase(c, s, scr, x1, idxp, valsp, out1, out_ui, None,
           n_ui, seg_len, n_seg, "last", 1.0 / 3.0)

  return pl.kernel(
      body,
      out_type=(_sds((NC * n_ui, HALF)),
                _sds((NC * n_ui, HALF), jnp.bfloat16),
                _sds((NC * n_ui, HALF), jnp.bfloat16)),
      mesh=_mesh(),
      compiler_params=pltpu.CompilerParams(use_tc_tiling_on_sc=False,
                                           needs_layout_passes=False),
      scratch_types=_scratch(n_ui, seg_len),
  )


def _make_k2(n_ui, nb, n_ub, seg_bi, nseg_bi, seg_ub, nseg_ub):
  def body(*refs):
    (out_ui, bi_idxp, bi_valsp, x0, ub_idxp, ub_valsp,
     out_bi, out1, x1, out_ub) = refs[:10]
    scr = _grab(refs, 10)
    c = lax.axis_index("c")
    s = lax.axis_index("s")
    _phase(c, s, scr, out_ui, bi_idxp, bi_valsp, None, out_bi, None,
           nb, seg_bi, nseg_bi, "plain")
    _phase(c, s, scr, x0, ub_idxp, ub_valsp, x0, out1, x1,
           n_ub, seg_ub, nseg_ub, "first")
    _phase(c, s, scr, x1, ub_idxp, ub_valsp, out1, out_ub, None,
           n_ub, seg_ub, nseg_ub, "last", 1.0 / 3.0)

  return pl.kernel(
      body,
      out_type=(_sds((NC * nb, HALF), jnp.bfloat16),
                _sds((NC * n_ub, HALF)),
                _sds((NC * n_ub, HALF), jnp.bfloat16),
                _sds((NC * n_ub, HALF), jnp.bfloat16)),
      mesh=_mesh(),
      compiler_params=pltpu.CompilerParams(use_tc_tiling_on_sc=False,
                                           needs_layout_passes=False),
      scratch_types=_scratch(n_ub, max(seg_bi, seg_ub)),
  )


def _make_score(batch):
  pb = batch // NS

  def body(t_ui, t_ub, t_bi, uidx_ui, uidx_ub, bidx_bi, bidx_ub, pred,
           ua, bb, g1, g2, g3, g4, ibuf, pbuf, sem):
    c = lax.axis_index("c")
    s = lax.axis_index("s")
    iota = lax.iota(jnp.int32, LN)

    def gath(idx_hbm, table, dst, n128):
      def g(j, _):
        pltpu.sync_copy(idx_hbm.at[c, pl.ds(s * n128 * CH + j * CH, CH)], ibuf)
        pltpu.async_copy(table.at[ibuf], dst.at[pl.ds(j * CH, CH)], sem).wait()
        return _
      lax.fori_loop(0, n128, g, 0)

    gath(uidx_ui, t_ui, g1, pb // CH)
    gath(uidx_ub, t_ub, g2, pb // CH)
    gath(bidx_bi, t_bi, g3, 2 * pb // CH)
    gath(bidx_ub, t_ub, g4, 2 * pb // CH)

    def upadd(dst, sa, sb, n):
      def go(r, _):
        a1, b1 = plsc.unpack(sa[r, pl.ds(0, 2 * LN)],
                             format=plsc.PackFormat.INTERLEAVED)
        a2, b2 = plsc.unpack(sb[r, pl.ds(0, 2 * LN)],
                             format=plsc.PackFormat.INTERLEAVED)
        dst[r, pl.ds(0, LN)] = a1 + a2
        dst[r, pl.ds(LN, LN)] = b1 + b2
        return _
      lax.fori_loop(0, n, go, 0, unroll=4)

    upadd(ua, g1, g2, pb)
    upadd(bb, g3, g4, 2 * pb)

    def dots(g, _):
      ru = g * LN + iota
      for k in range(2):
        rb = ru * 2 + k
        d = jnp.zeros((LN,), jnp.float32)
        for j in range(HALF):
          cj = jnp.full((LN,), j, jnp.int32)
          a = plsc.load_gather(ua, [ru, cj])
          b = plsc.load_gather(bb, [rb, cj])
          d = d + a * b
        pbuf[k, pl.ds(g * LN, LN)] = d
      return _
    lax.fori_loop(0, pb // LN, dots, 0)

    def wout(k, _):
      pltpu.sync_copy(pbuf.at[k], pred.at[c * 2 + k, pl.ds(s * pb, pb)])
      return _
    lax.fori_loop(0, 2, wout, 0)

  return pl.kernel(
      body,
      out_type=_sds((NC * 2, batch)),
      mesh=_mesh(),
      compiler_params=pltpu.CompilerParams(use_tc_tiling_on_sc=False, needs_layout_passes=False),
      scratch_types=[
          pltpu.VMEM((pb, HALF), jnp.float32),
          pltpu.VMEM((2 * pb, HALF), jnp.float32),
          pltpu.VMEM((pb, 2 * LN), jnp.bfloat16),
          pltpu.VMEM((pb, 2 * LN), jnp.bfloat16),
          pltpu.VMEM((2 * pb, 2 * LN), jnp.bfloat16),
          pltpu.VMEM((2 * pb, 2 * LN), jnp.bfloat16),
          pltpu.VMEM((CH,), jnp.int32),
          pltpu.VMEM((2, pb), jnp.float32),
          pltpu.SemaphoreType.DMA,
      ],
  )


def _split_cols(x):
  n = x.shape[0]
  return x.reshape(n, NC, HALF).transpose(1, 0, 2).reshape(NC * n, HALF)


def _pack_edges(src, dst, vals, n_src, seg_len, n_seg):
  e = src.shape[0]
  n_chunks = seg_len * n_seg
  pad = NS * CH * n_chunks - e
  assert pad >= 0
  if pad:
    src = jnp.pad(src, (0, pad))
    dst = jnp.pad(dst, (0, pad))
    vals = jnp.pad(vals, (0, pad))
  sd = jnp.stack([src, dst], axis=0).reshape(2, NS, n_chunks, CH)
  idxp = jnp.stack(
      [sd.at[0].add(c * n_src) for c in range(NC)],
      axis=0).transpose(0, 2, 3, 1, 4)
  idxp = idxp.reshape(NC, NS, n_seg, seg_len, 2, CH)
  valsp = vals.reshape(NS, n_seg, seg_len * CH)
  return idxp, valsp


def kernel(users_feature, items_feature, bundles_feature,
           ui_rows, ui_cols, ui_vals,
           bi_rows, bi_cols, bi_vals,
           ub_rows, ub_cols, ub_vals,
           users_idx, bundles_idx):
  u, i, nb = _U, _I, _NB
  n_ui = u + i
  n_ub = u + nb

  ui_rows = ui_rows.astype(jnp.int32)
  ui_cols = ui_cols.astype(jnp.int32)
  bi_rows = bi_rows.astype(jnp.int32)
  bi_cols = bi_cols.astype(jnp.int32)
  ub_rows = ub_rows.astype(jnp.int32)
  ub_cols = ub_cols.astype(jnp.int32)

  x0_ui = _split_cols(
      jnp.concatenate([users_feature, items_feature], axis=0)
  ).astype(jnp.bfloat16)
  x0_ub = _split_cols(
      jnp.concatenate([users_feature, bundles_feature], axis=0)
  ).astype(jnp.bfloat16)

  ui_seg, ui_nseg = 40, 16
  bi_seg, bi_nseg = 40, 4
  ub_seg, ub_nseg = 40, 5

  ui_idx, ui_v = _pack_edges(
      jnp.concatenate([ui_rows, ui_cols + u]),
      jnp.concatenate([ui_cols + u, ui_rows]),
      jnp.concatenate([ui_vals, ui_vals]), n_ui, ui_seg, ui_nseg)
  ub_idx, ub_v = _pack_edges(
      jnp.concatenate([ub_rows, ub_cols + u]),
      jnp.concatenate([ub_cols + u, ub_rows]),
      jnp.concatenate([ub_vals, ub_vals]), n_ub, ub_seg, ub_nseg)
  bi_idx, bi_v = _pack_edges(bi_cols + u, bi_rows, bi_vals, n_ui,
                             bi_seg, bi_nseg)

  _, _, out_ui = _make_k1(n_ui, ui_seg, ui_nseg)(x0_ui, ui_idx, ui_v)
  out_bi, _, _, out_ub = _make_k2(n_ui, nb, n_ub, bi_seg, bi_nseg,
                                  ub_seg, ub_nseg)(
      out_ui, bi_idx, bi_v, x0_ub, ub_idx, ub_v)

  batch = users_idx.shape[0]
  uix = users_idx.astype(jnp.int32)
  bix = bundles_idx.astype(jnp.int32).reshape(-1)
  uidx_ui = jnp.stack([uix + c * n_ui for c in range(NC)], axis=0)
  uidx_ub = jnp.stack([uix + c * n_ub for c in range(NC)], axis=0)
  bidx_bi = jnp.stack([bix + c * nb for c in range(NC)], axis=0)
  bidx_ub = jnp.stack([bix + u + c * n_ub for c in range(NC)], axis=0)

  pred4 = _make_score(batch)(
      out_ui, out_ub, out_bi, uidx_ui, uidx_ub, bidx_bi, bidx_ub)
  return jnp.stack([pred4[0] + pred4[2], pred4[1] + pred4[3]], axis=-1)


# speedup vs baseline: 1.3466x; 1.3431x over previous
import functools

import jax
import jax.numpy as jnp
from jax import lax
from jax.experimental import pallas as pl
from jax.experimental.pallas import tpu as pltpu
from jax.experimental.pallas import tpu_sc as plsc

NC = 2
NS = 16
LN = 16
HALF = 32
CH = 128
DR = 125

_U, _I, _NB = 20000, 20000, 10000


def _mesh():
  return plsc.VectorSubcoreMesh(
      core_axis_name="c", subcore_axis_name="s", num_cores=NC, num_subcores=NS)


def _sds(shape, dtype=jnp.float32):
  return jax.ShapeDtypeStruct(shape, dtype)


NBUF = 4
LOOK = 3


def _phase(c, s, scr, table, idxp, valsp, add_into, out, x_next,
           n_out, seg_len, n_seg, mode, scale=1.0):
  (acc, ibig, vbig, rows_buf, sbuf, d1, d2, d1b, d2b, gsems, ssems) = scr
  rpt = n_out // NS
  assert rpt % DR == 0 and seg_len % NBUF == 0
  zvec = jnp.zeros((LN,), jnp.float32)

  def zrow(r, _):
    d1[r, pl.ds(0, LN)] = zvec
    d1[r, pl.ds(LN, LN)] = zvec
    return _
  lax.fori_loop(0, DR, zrow, 0)

  def zchunk(t, _):
    pltpu.sync_copy(d1, acc.at[pl.ds(s * rpt + t * DR, DR)])
    return _
  lax.fori_loop(0, rpt // DR, zchunk, 0)
  plsc.subcore_barrier()

  def gwait(j, k):
    pltpu.make_async_copy(table.at[ibig.at[j, 0]], rows_buf.at[k],
                          gsems[k]).wait()

  def gstart(j, k):
    pltpu.async_copy(table.at[ibig.at[j, 0]], rows_buf.at[k], gsems[k])

  def sstart(j, kp):
    pltpu.async_copy(sbuf.at[kp], acc.at[ibig.at[j, 1]], ssems[kp],
                     add=True)

  def swait(j, kp):
    pltpu.make_async_copy(sbuf.at[kp], acc.at[ibig.at[j, 1]],
                          ssems[kp]).wait()

  def segment(seg, _):
    pltpu.sync_copy(idxp.at[c, s, seg], ibig)
    pltpu.sync_copy(valsp.at[s, seg], vbig)
    for k in range(LOOK):
      gstart(k, k)

    def round_(g, _):
      for k in range(NBUF):
        j = g * NBUF + k
        kp = k % 2
        gwait(j, k)

        @pl.when(j + LOOK < seg_len)
        def _issue_next():
          gstart(j + LOOK, (k + LOOK) % NBUF)

        @pl.when(j >= 2)
        def _wait_prev():
          swait(j - 2, kp)

        vbase = jnp.full((LN,), j * CH, jnp.int32)

        @plsc.parallel_loop(0, CH, unroll=8)
        def _scl(e):
          v = plsc.load_gather(vbig, [vbase + e])
          a, b = plsc.unpack(rows_buf[k, e, pl.ds(0, 2 * LN)],
                             format=plsc.PackFormat.INTERLEAVED)
          sbuf[kp, e, pl.ds(0, LN)] = a * v
          sbuf[kp, e, pl.ds(LN, LN)] = b * v

        sstart(j, kp)
      return _
    lax.fori_loop(0, seg_len // NBUF, round_, 0)
    for kp in range(2):
      swait(0, kp)
    return _
  lax.fori_loop(0, n_seg, segment, 0)
  plsc.subcore_barrier()

  def pack_rows(src_f32):
    def pk(r, _):
      d1b[r, pl.ds(0, 2 * LN)] = plsc.pack(
          src_f32[r, pl.ds(0, LN)], src_f32[r, pl.ds(LN, LN)],
          format=plsc.PackFormat.INTERLEAVED)
      return _
    lax.fori_loop(0, DR, pk, 0, unroll=4)

  def dchunk(t, _):
    r0 = s * rpt + t * DR
    g0 = c * n_out + r0
    pltpu.sync_copy(acc.at[pl.ds(r0, DR)], d1)
    if mode == "first":
      pack_rows(d1)
      pltpu.sync_copy(d1b, x_next.at[pl.ds(g0, DR)])
      pltpu.sync_copy(add_into.at[pl.ds(g0, DR)], d2b)

      def addv(r, _):
        a, b = plsc.unpack(d2b[r, pl.ds(0, 2 * LN)],
                           format=plsc.PackFormat.INTERLEAVED)
        d1[r, pl.ds(0, LN)] = d1[r, pl.ds(0, LN)] + a
        d1[r, pl.ds(LN, LN)] = d1[r, pl.ds(LN, LN)] + b
        return _
      lax.fori_loop(0, DR, addv, 0, unroll=4)
      pltpu.sync_copy(d1, out.at[pl.ds(g0, DR)])
    elif mode == "last":
      pltpu.sync_copy(add_into.at[pl.ds(g0, DR)], d2)

      def addv(r, _):
        d1[r, pl.ds(0, LN)] = (d1[r, pl.ds(0, LN)] +
                               d2[r, pl.ds(0, LN)]) * scale
        d1[r, pl.ds(LN, LN)] = (d1[r, pl.ds(LN, LN)] +
                                d2[r, pl.ds(LN, LN)]) * scale
        return _
      lax.fori_loop(0, DR, addv, 0, unroll=4)
      pack_rows(d1)
      pltpu.sync_copy(d1b, out.at[pl.ds(g0, DR)])
    else:
      pack_rows(d1)
      pltpu.sync_copy(d1b, out.at[pl.ds(g0, DR)])
    return _
  lax.fori_loop(0, rpt // DR, dchunk, 0)
  plsc.subcore_barrier()


def _scratch(n_acc, seg_len):
  return [
      pltpu.VMEM_SHARED((n_acc, HALF), jnp.float32),
      pltpu.VMEM((seg_len, 2, CH), jnp.int32),
      pltpu.VMEM((seg_len * CH,), jnp.float32),
      pltpu.VMEM((NBUF, CH, 2 * LN), jnp.bfloat16),
      pltpu.VMEM((2, CH, HALF), jnp.float32),
      pltpu.VMEM((DR, HALF), jnp.float32),
      pltpu.VMEM((DR, HALF), jnp.float32),
      pltpu.VMEM((DR, 2 * LN), jnp.bfloat16),
      pltpu.VMEM((DR, 2 * LN), jnp.bfloat16),
  ] + [pltpu.SemaphoreType.DMA] * (NBUF + 2)


def _grab(refs, n_io):
  scr = list(refs[n_io:])
  return tuple(scr[:9]) + (scr[9:9 + NBUF], scr[9 + NBUF:])


def _make_k1(n_ui, seg_len, n_seg):
  def body(*refs):
    (x0, idxp, valsp, out1, x1, out_ui) = refs[:6]
    scr = _grab(refs, 6)
    c = lax.axis_index("c")
    s = lax.axis_index("s")
    _phase(c, s, scr, x0, idxp, valsp, x0, out1, x1,
           n_ui, seg_len, n_seg, "first")
    _phase(c, s, scr, x1, idxp, valsp, out1, out_ui, None,
           n_ui, seg_len, n_seg, "last", 1.0 / 3.0)

  return pl.kernel(
      body,
      out_type=(_sds((NC * n_ui, HALF)),
                _sds((NC * n_ui, HALF), jnp.bfloat16),
                _sds((NC * n_ui, HALF), jnp.bfloat16)),
      mesh=_mesh(),
      compiler_params=pltpu.CompilerParams(use_tc_tiling_on_sc=False,
                                           needs_layout_passes=False),
      scratch_types=_scratch(n_ui, seg_len),
  )


def _make_k2(n_ui, nb, n_ub, seg_bi, nseg_bi, seg_ub, nseg_ub):
  def body(*refs):
    (out_ui, bi_idxp, bi_valsp, x0, ub_idxp, ub_valsp,
     out_bi, out1, x1, out_ub) = refs[:10]
    scr = _grab(refs, 10)
    c = lax.axis_index("c")
    s = lax.axis_index("s")
    _phase(c, s, scr, out_ui, bi_idxp, bi_valsp, None, out_bi, None,
           nb, seg_bi, nseg_bi, "plain")
    _phase(c, s, scr, x0, ub_idxp, ub_valsp, x0, out1, x1,
           n_ub, seg_ub, nseg_ub, "first")
    _phase(c, s, scr, x1, ub_idxp, ub_valsp, out1, out_ub, None,
           n_ub, seg_ub, nseg_ub, "last", 1.0 / 3.0)

  return pl.kernel(
      body,
      out_type=(_sds((NC * nb, HALF), jnp.bfloat16),
                _sds((NC * n_ub, HALF)),
                _sds((NC * n_ub, HALF), jnp.bfloat16),
                _sds((NC * n_ub, HALF), jnp.bfloat16)),
      mesh=_mesh(),
      compiler_params=pltpu.CompilerParams(use_tc_tiling_on_sc=False,
                                           needs_layout_passes=False),
      scratch_types=_scratch(n_ub, max(seg_bi, seg_ub)),
  )


def _make_score(batch):
  pb = batch // NS

  def body(t_ui, t_ub, t_bi, uidx_ui, uidx_ub, bidx_bi, bidx_ub, pred,
           ua, bb, g1, g2, g3, g4, ibuf, pbuf, sem):
    c = lax.axis_index("c")
    s = lax.axis_index("s")
    iota = lax.iota(jnp.int32, LN)

    def gath(idx_hbm, table, dst, n128):
      def g(j, _):
        pltpu.sync_copy(idx_hbm.at[c, pl.ds(s * n128 * CH + j * CH, CH)], ibuf)
        pltpu.async_copy(table.at[ibuf], dst.at[pl.ds(j * CH, CH)], sem).wait()
        return _
      lax.fori_loop(0, n128, g, 0)

    gath(uidx_ui, t_ui, g1, pb // CH)
    gath(uidx_ub, t_ub, g2, pb // CH)
    gath(bidx_bi, t_bi, g3, 2 * pb // CH)
    gath(bidx_ub, t_ub, g4, 2 * pb // CH)

    def upadd(dst, sa, sb, n):
      def go(r, _):
        a1, b1 = plsc.unpack(sa[r, pl.ds(0, 2 * LN)],
                             format=plsc.PackFormat.INTERLEAVED)
        a2, b2 = plsc.unpack(sb[r, pl.ds(0, 2 * LN)],
                             format=plsc.PackFormat.INTERLEAVED)
        dst[r, pl.ds(0, LN)] = a1 + a2
        dst[r, pl.ds(LN, LN)] = b1 + b2
        return _
      lax.fori_loop(0, n, go, 0, unroll=4)

    upadd(ua, g1, g2, pb)
    upadd(bb, g3, g4, 2 * pb)

    def dots(g, _):
      ru = g * LN + iota
      for k in range(2):
        rb = ru * 2 + k
        d = jnp.zeros((LN,), jnp.float32)
        for j in range(HALF):
          cj = jnp.full((LN,), j, jnp.int32)
          a = plsc.load_gather(ua, [ru, cj])
          b = plsc.load_gather(bb, [rb, cj])
          d = d + a * b
        pbuf[k, pl.ds(g * LN, LN)] = d
      return _
    lax.fori_loop(0, pb // LN, dots, 0)

    def wout(k, _):
      pltpu.sync_copy(pbuf.at[k], pred.at[c * 2 + k, pl.ds(s * pb, pb)])
      return _
    lax.fori_loop(0, 2, wout, 0)

  return pl.kernel(
      body,
      out_type=_sds((NC * 2, batch)),
      mesh=_mesh(),
      compiler_params=pltpu.CompilerParams(use_tc_tiling_on_sc=False, needs_layout_passes=False),
      scratch_types=[
          pltpu.VMEM((pb, HALF), jnp.float32),
          pltpu.VMEM((2 * pb, HALF), jnp.float32),
          pltpu.VMEM((pb, 2 * LN), jnp.bfloat16),
          pltpu.VMEM((pb, 2 * LN), jnp.bfloat16),
          pltpu.VMEM((2 * pb, 2 * LN), jnp.bfloat16),
          pltpu.VMEM((2 * pb, 2 * LN), jnp.bfloat16),
          pltpu.VMEM((CH,), jnp.int32),
          pltpu.VMEM((2, pb), jnp.float32),
          pltpu.SemaphoreType.DMA,
      ],
  )


def _split_cols(x):
  n = x.shape[0]
  return x.reshape(n, NC, HALF).transpose(1, 0, 2).reshape(NC * n, HALF)


def _pack_edges(src, dst, vals, n_src, seg_len, n_seg):
  e = src.shape[0]
  n_chunks = seg_len * n_seg
  pad = NS * CH * n_chunks - e
  assert pad >= 0
  if pad:
    src = jnp.pad(src, (0, pad))
    dst = jnp.pad(dst, (0, pad))
    vals = jnp.pad(vals, (0, pad))
  sd = jnp.stack([src, dst], axis=0).reshape(2, NS, n_chunks, CH)
  idxp = jnp.stack(
      [sd.at[0].add(c * n_src) for c in range(NC)],
      axis=0).transpose(0, 2, 3, 1, 4)
  idxp = idxp.reshape(NC, NS, n_seg, seg_len, 2, CH)
  valsp = vals.reshape(NS, n_seg, seg_len * CH)
  return idxp, valsp


def kernel(users_feature, items_feature, bundles_feature,
           ui_rows, ui_cols, ui_vals,
           bi_rows, bi_cols, bi_vals,
           ub_rows, ub_cols, ub_vals,
           users_idx, bundles_idx):
  u, i, nb = _U, _I, _NB
  n_ui = u + i
  n_ub = u + nb

  ui_rows = ui_rows.astype(jnp.int32)
  ui_cols = ui_cols.astype(jnp.int32)
  bi_rows = bi_rows.astype(jnp.int32)
  bi_cols = bi_cols.astype(jnp.int32)
  ub_rows = ub_rows.astype(jnp.int32)
  ub_cols = ub_cols.astype(jnp.int32)

  x0_ui = _split_cols(
      jnp.concatenate([users_feature, items_feature], axis=0)
  ).astype(jnp.bfloat16)
  x0_ub = _split_cols(
      jnp.concatenate([users_feature, bundles_feature], axis=0)
  ).astype(jnp.bfloat16)

  ui_seg, ui_nseg = 40, 16
  bi_seg, bi_nseg = 40, 4
  ub_seg, ub_nseg = 40, 5

  ui_idx, ui_v = _pack_edges(
      jnp.concatenate([ui_rows, ui_cols + u]),
      jnp.concatenate([ui_cols + u, ui_rows]),
      jnp.concatenate([ui_vals, ui_vals]), n_ui, ui_seg, ui_nseg)
  ub_idx, ub_v = _pack_edges(
      jnp.concatenate([ub_rows, ub_cols + u]),
      jnp.concatenate([ub_cols + u, ub_rows]),
      jnp.concatenate([ub_vals, ub_vals]), n_ub, ub_seg, ub_nseg)
  bi_idx, bi_v = _pack_edges(bi_cols + u, bi_rows, bi_vals, n_ui,
                             bi_seg, bi_nseg)

  _, _, out_ui = _make_k1(n_ui, ui_seg, ui_nseg)(x0_ui, ui_idx, ui_v)
  out_bi, _, _, out_ub = _make_k2(n_ui, nb, n_ub, bi_seg, bi_nseg,
                                  ub_seg, ub_nseg)(
      out_ui, bi_idx, bi_v, x0_ub, ub_idx, ub_v)

  batch = users_idx.shape[0]
  uix = users_idx.astype(jnp.int32)
  bix = bundles_idx.astype(jnp.int32).reshape(-1)
  uidx_ui = jnp.stack([uix + c * n_ui for c in range(NC)], axis=0)
  uidx_ub = jnp.stack([uix + c * n_ub for c in range(NC)], axis=0)
  bidx_bi = jnp.stack([bix + c * nb for c in range(NC)], axis=0)
  bidx_ub = jnp.stack([bix + u + c * n_ub for c in range(NC)], axis=0)

  pred4 = _make_score(batch)(
      out_ui, out_ub, out_bi, uidx_ui, uidx_ub, bidx_bi, bidx_ub)
  return jnp.stack([pred4[0] + pred4[2], pred4[1] + pred4[3]], axis=-1)


# speedup vs baseline: 1.4150x; 1.0508x over previous
import functools

import jax
import jax.numpy as jnp
from jax import lax
from jax.experimental import pallas as pl
from jax.experimental.pallas import tpu as pltpu
from jax.experimental.pallas import tpu_sc as plsc

NC = 2
NS = 16
LN = 16
HALF = 32
CH = 128
DR = 125

_U, _I, _NB = 20000, 20000, 10000


def _mesh():
  return plsc.VectorSubcoreMesh(
      core_axis_name="c", subcore_axis_name="s", num_cores=NC, num_subcores=NS)


def _sds(shape, dtype=jnp.float32):
  return jax.ShapeDtypeStruct(shape, dtype)


NBUF = 4
LOOK = 3


def _phase(c, s, scr, table, idxp, valsp, add_into, out, x_next,
           n_out, seg_len, n_seg, mode, scale=1.0):
  (acc, ibig, vbig, rows_buf, sbuf, d1, d2, d1b, d2b, gsems, ssems) = scr
  rpt = n_out // NS
  assert rpt % DR == 0 and seg_len % NBUF == 0
  zvec = jnp.zeros((LN,), jnp.float32)

  @plsc.parallel_loop(0, DR, unroll=8)
  def _zrow(r):
    d1[r, pl.ds(0, LN)] = zvec
    d1[r, pl.ds(LN, LN)] = zvec

  def zchunk(t, _):
    pltpu.sync_copy(d1, acc.at[pl.ds(s * rpt + t * DR, DR)])
    return _
  lax.fori_loop(0, rpt // DR, zchunk, 0)
  plsc.subcore_barrier()

  def gwait(j, k):
    pltpu.make_async_copy(table.at[ibig.at[j, 0]], rows_buf.at[k],
                          gsems[k]).wait()

  def gstart(j, k):
    pltpu.async_copy(table.at[ibig.at[j, 0]], rows_buf.at[k], gsems[k])

  def sstart(j, kp):
    pltpu.async_copy(sbuf.at[kp], acc.at[ibig.at[j, 1]], ssems[kp],
                     add=True)

  def swait(j, kp):
    pltpu.make_async_copy(sbuf.at[kp], acc.at[ibig.at[j, 1]],
                          ssems[kp]).wait()

  def segment(seg, _):
    pltpu.sync_copy(idxp.at[c, s, seg], ibig)
    pltpu.sync_copy(valsp.at[s, seg], vbig)
    for k in range(LOOK):
      gstart(k, k)

    def round_(g, _):
      for k in range(NBUF):
        j = g * NBUF + k
        kp = k % 2
        gwait(j, k)

        @pl.when(j + LOOK < seg_len)
        def _issue_next():
          gstart(j + LOOK, (k + LOOK) % NBUF)

        @pl.when(j >= 2)
        def _wait_prev():
          swait(j - 2, kp)

        vbase = jnp.full((LN,), j * CH, jnp.int32)

        @plsc.parallel_loop(0, CH, unroll=16)
        def _scl(e):
          v = plsc.load_gather(vbig, [vbase + e])
          a, b = plsc.unpack(rows_buf[k, e, pl.ds(0, 2 * LN)],
                             format=plsc.PackFormat.INTERLEAVED)
          sbuf[kp, e, pl.ds(0, LN)] = a * v
          sbuf[kp, e, pl.ds(LN, LN)] = b * v

        sstart(j, kp)
      return _
    lax.fori_loop(0, seg_len // NBUF, round_, 0)
    for kp in range(2):
      swait(0, kp)
    return _
  lax.fori_loop(0, n_seg, segment, 0)
  plsc.subcore_barrier()

  def pack_rows(src_f32):
    @plsc.parallel_loop(0, DR, unroll=8)
    def _pk(r):
      d1b[r, pl.ds(0, 2 * LN)] = plsc.pack(
          src_f32[r, pl.ds(0, LN)], src_f32[r, pl.ds(LN, LN)],
          format=plsc.PackFormat.INTERLEAVED)

  def dchunk(t, _):
    r0 = s * rpt + t * DR
    g0 = c * n_out + r0
    pltpu.sync_copy(acc.at[pl.ds(r0, DR)], d1)
    if mode == "first":
      pack_rows(d1)
      pltpu.sync_copy(d1b, x_next.at[pl.ds(g0, DR)])
      pltpu.sync_copy(add_into.at[pl.ds(g0, DR)], d2b)

      @plsc.parallel_loop(0, DR, unroll=8)
      def _addv(r):
        a, b = plsc.unpack(d2b[r, pl.ds(0, 2 * LN)],
                           format=plsc.PackFormat.INTERLEAVED)
        d1[r, pl.ds(0, LN)] = d1[r, pl.ds(0, LN)] + a
        d1[r, pl.ds(LN, LN)] = d1[r, pl.ds(LN, LN)] + b
      pltpu.sync_copy(d1, out.at[pl.ds(g0, DR)])
    elif mode == "last":
      pltpu.sync_copy(add_into.at[pl.ds(g0, DR)], d2)

      @plsc.parallel_loop(0, DR, unroll=8)
      def _addv(r):
        d1[r, pl.ds(0, LN)] = (d1[r, pl.ds(0, LN)] +
                               d2[r, pl.ds(0, LN)]) * scale
        d1[r, pl.ds(LN, LN)] = (d1[r, pl.ds(LN, LN)] +
                                d2[r, pl.ds(LN, LN)]) * scale
      pack_rows(d1)
      pltpu.sync_copy(d1b, out.at[pl.ds(g0, DR)])
    else:
      pack_rows(d1)
      pltpu.sync_copy(d1b, out.at[pl.ds(g0, DR)])
    return _
  lax.fori_loop(0, rpt // DR, dchunk, 0)
  plsc.subcore_barrier()


def _scratch(n_acc, seg_len):
  return [
      pltpu.VMEM_SHARED((n_acc, HALF), jnp.float32),
      pltpu.VMEM((seg_len, 2, CH), jnp.int32),
      pltpu.VMEM((seg_len * CH,), jnp.float32),
      pltpu.VMEM((NBUF, CH, 2 * LN), jnp.bfloat16),
      pltpu.VMEM((2, CH, HALF), jnp.float32),
      pltpu.VMEM((DR, HALF), jnp.float32),
      pltpu.VMEM((DR, HALF), jnp.float32),
      pltpu.VMEM((DR, 2 * LN), jnp.bfloat16),
      pltpu.VMEM((DR, 2 * LN), jnp.bfloat16),
  ] + [pltpu.SemaphoreType.DMA] * (NBUF + 2)


def _grab(refs, n_io):
  scr = list(refs[n_io:])
  return tuple(scr[:9]) + (scr[9:9 + NBUF], scr[9 + NBUF:])


def _make_k1(n_ui, seg_len, n_seg):
  def body(*refs):
    (x0, idxp, valsp, out1, x1, out_ui) = refs[:6]
    scr = _grab(refs, 6)
    c = lax.axis_index("c")
    s = lax.axis_index("s")
    _phase(c, s, scr, x0, idxp, valsp, x0, out1, x1,
           n_ui, seg_len, n_seg, "first")
    _phase(c, s, scr, x1, idxp, valsp, out1, out_ui, None,
           n_ui, seg_len, n_seg, "last", 1.0 / 3.0)

  return pl.kernel(
      body,
      out_type=(_sds((NC * n_ui, HALF)),
                _sds((NC * n_ui, HALF), jnp.bfloat16),
                _sds((NC * n_ui, HALF), jnp.bfloat16)),
      mesh=_mesh(),
      compiler_params=pltpu.CompilerParams(use_tc_tiling_on_sc=False,
                                           needs_layout_passes=False),
      scratch_types=_scratch(n_ui, seg_len),
  )


def _make_k2(n_ui, nb, n_ub, seg_bi, nseg_bi, seg_ub, nseg_ub):
  def body(*refs):
    (out_ui, bi_idxp, bi_valsp, x0, ub_idxp, ub_valsp,
     out_bi, out1, x1, out_ub) = refs[:10]
    scr = _grab(refs, 10)
    c = lax.axis_index("c")
    s = lax.axis_index("s")
    _phase(c, s, scr, out_ui, bi_idxp, bi_valsp, None, out_bi, None,
           nb, seg_bi, nseg_bi, "plain")
    _phase(c, s, scr, x0, ub_idxp, ub_valsp, x0, out1, x1,
           n_ub, seg_ub, nseg_ub, "first")
    _phase(c, s, scr, x1, ub_idxp, ub_valsp, out1, out_ub, None,
           n_ub, seg_ub, nseg_ub, "last", 1.0 / 3.0)

  return pl.kernel(
      body,
      out_type=(_sds((NC * nb, HALF), jnp.bfloat16),
                _sds((NC * n_ub, HALF)),
                _sds((NC * n_ub, HALF), jnp.bfloat16),
                _sds((NC * n_ub, HALF), jnp.bfloat16)),
      mesh=_mesh(),
      compiler_params=pltpu.CompilerParams(use_tc_tiling_on_sc=False,
                                           needs_layout_passes=False),
      scratch_types=_scratch(n_ub, max(seg_bi, seg_ub)),
  )


def _make_score(batch):
  pb = batch // NS

  def body(t_ui, t_ub, t_bi, uidx_ui, uidx_ub, bidx_bi, bidx_ub, pred,
           ua, bb, g1, g2, g3, g4, ibuf, pbuf, sem):
    c = lax.axis_index("c")
    s = lax.axis_index("s")
    iota = lax.iota(jnp.int32, LN)

    def gath(idx_hbm, table, dst, n128):
      def g(j, _):
        pltpu.sync_copy(idx_hbm.at[c, pl.ds(s * n128 * CH + j * CH, CH)], ibuf)
        pltpu.async_copy(table.at[ibuf], dst.at[pl.ds(j * CH, CH)], sem).wait()
        return _
      lax.fori_loop(0, n128, g, 0)

    gath(uidx_ui, t_ui, g1, pb // CH)
    gath(uidx_ub, t_ub, g2, pb // CH)
    gath(bidx_bi, t_bi, g3, 2 * pb // CH)
    gath(bidx_ub, t_ub, g4, 2 * pb // CH)

    def upadd(dst, sa, sb, n):
      @plsc.parallel_loop(0, n, unroll=8)
      def _go(r):
        a1, b1 = plsc.unpack(sa[r, pl.ds(0, 2 * LN)],
                             format=plsc.PackFormat.INTERLEAVED)
        a2, b2 = plsc.unpack(sb[r, pl.ds(0, 2 * LN)],
                             format=plsc.PackFormat.INTERLEAVED)
        dst[r, pl.ds(0, LN)] = a1 + a2
        dst[r, pl.ds(LN, LN)] = b1 + b2

    upadd(ua, g1, g2, pb)
    upadd(bb, g3, g4, 2 * pb)

    def dots(g, _):
      ru = g * LN + iota
      for k in range(2):
        rb = ru * 2 + k
        d = jnp.zeros((LN,), jnp.float32)
        for j in range(HALF):
          cj = jnp.full((LN,), j, jnp.int32)
          a = plsc.load_gather(ua, [ru, cj])
          b = plsc.load_gather(bb, [rb, cj])
          d = d + a * b
        pbuf[k, pl.ds(g * LN, LN)] = d
      return _
    lax.fori_loop(0, pb // LN, dots, 0)

    def wout(k, _):
      pltpu.sync_copy(pbuf.at[k], pred.at[c * 2 + k, pl.ds(s * pb, pb)])
      return _
    lax.fori_loop(0, 2, wout, 0)

  return pl.kernel(
      body,
      out_type=_sds((NC * 2, batch)),
      mesh=_mesh(),
      compiler_params=pltpu.CompilerParams(use_tc_tiling_on_sc=False, needs_layout_passes=False),
      scratch_types=[
          pltpu.VMEM((pb, HALF), jnp.float32),
          pltpu.VMEM((2 * pb, HALF), jnp.float32),
          pltpu.VMEM((pb, 2 * LN), jnp.bfloat16),
          pltpu.VMEM((pb, 2 * LN), jnp.bfloat16),
          pltpu.VMEM((2 * pb, 2 * LN), jnp.bfloat16),
          pltpu.VMEM((2 * pb, 2 * LN), jnp.bfloat16),
          pltpu.VMEM((CH,), jnp.int32),
          pltpu.VMEM((2, pb), jnp.float32),
          pltpu.SemaphoreType.DMA,
      ],
  )


def _split_cols(x):
  n = x.shape[0]
  return x.reshape(n, NC, HALF).transpose(1, 0, 2).reshape(NC * n, HALF)


def _pack_edges(src, dst, vals, n_src, seg_len, n_seg):
  e = src.shape[0]
  n_chunks = seg_len * n_seg
  pad = NS * CH * n_chunks - e
  assert pad >= 0
  if pad:
    src = jnp.pad(src, (0, pad))
    dst = jnp.pad(dst, (0, pad))
    vals = jnp.pad(vals, (0, pad))
  sd = jnp.stack([src, dst], axis=0).reshape(2, NS, n_chunks, CH)
  idxp = jnp.stack(
      [sd.at[0].add(c * n_src) for c in range(NC)],
      axis=0).transpose(0, 2, 3, 1, 4)
  idxp = idxp.reshape(NC, NS, n_seg, seg_len, 2, CH)
  valsp = vals.reshape(NS, n_seg, seg_len * CH)
  return idxp, valsp


def kernel(users_feature, items_feature, bundles_feature,
           ui_rows, ui_cols, ui_vals,
           bi_rows, bi_cols, bi_vals,
           ub_rows, ub_cols, ub_vals,
           users_idx, bundles_idx):
  u, i, nb = _U, _I, _NB
  n_ui = u + i
  n_ub = u + nb

  ui_rows = ui_rows.astype(jnp.int32)
  ui_cols = ui_cols.astype(jnp.int32)
  bi_rows = bi_rows.astype(jnp.int32)
  bi_cols = bi_cols.astype(jnp.int32)
  ub_rows = ub_rows.astype(jnp.int32)
  ub_cols = ub_cols.astype(jnp.int32)

  x0_ui = _split_cols(
      jnp.concatenate([users_feature, items_feature], axis=0)
  ).astype(jnp.bfloat16)
  x0_ub = _split_cols(
      jnp.concatenate([users_feature, bundles_feature], axis=0)
  ).astype(jnp.bfloat16)

  ui_seg, ui_nseg = 40, 16
  bi_seg, bi_nseg = 40, 4
  ub_seg, ub_nseg = 40, 5

  ui_idx, ui_v = _pack_edges(
      jnp.concatenate([ui_rows, ui_cols + u]),
      jnp.concatenate([ui_cols + u, ui_rows]),
      jnp.concatenate([ui_vals, ui_vals]), n_ui, ui_seg, ui_nseg)
  ub_idx, ub_v = _pack_edges(
      jnp.concatenate([ub_rows, ub_cols + u]),
      jnp.concatenate([ub_cols + u, ub_rows]),
      jnp.concatenate([ub_vals, ub_vals]), n_ub, ub_seg, ub_nseg)
  bi_idx, bi_v = _pack_edges(bi_cols + u, bi_rows, bi_vals, n_ui,
                             bi_seg, bi_nseg)

  _, _, out_ui = _make_k1(n_ui, ui_seg, ui_nseg)(x0_ui, ui_idx, ui_v)
  out_bi, _, _, out_ub = _make_k2(n_ui, nb, n_ub, bi_seg, bi_nseg,
                                  ub_seg, ub_nseg)(
      out_ui, bi_idx, bi_v, x0_ub, ub_idx, ub_v)

  batch = users_idx.shape[0]
  uix = users_idx.astype(jnp.int32)
  bix = bundles_idx.astype(jnp.int32).reshape(-1)
  uidx_ui = jnp.stack([uix + c * n_ui for c in range(NC)], axis=0)
  uidx_ub = jnp.stack([uix + c * n_ub for c in range(NC)], axis=0)
  bidx_bi = jnp.stack([bix + c * nb for c in range(NC)], axis=0)
  bidx_ub = jnp.stack([bix + u + c * n_ub for c in range(NC)], axis=0)

  pred4 = _make_score(batch)(
      out_ui, out_ub, out_bi, uidx_ui, uidx_ub, bidx_bi, bidx_ub)
  return jnp.stack([pred4[0] + pred4[2], pred4[1] + pred4[3]], axis=-1)
